# Initial kernel scaffold; baseline (speedup 1.0000x reference)
import jax
import jax.numpy as jnp
from jax.experimental import pallas as pl


def kernel(x, edge_index, W1, b1, g1, be1, W2, b2, g2, be2, W3, b3):
    raise NotImplementedError("write your pallas kernel here")


# speedup vs baseline: 12.6713x; 12.6713x over previous
import jax
import jax.numpy as jnp
from jax import lax
from jax.experimental import pallas as pl
from jax.experimental.pallas import tpu as pltpu
from jax.experimental.pallas import tpu_sc as plsc

N = 10000
NPAD = 10112
E = 320000
CH = 128
NC, NS = 2, 16
NW = NC * NS
K = -(-E // (NW * CH))
EPT = K * CH
EPAD = EPT * NW
STRIPE = NPAD // NS
BN_EPS = 1e-5
BLK = 1000


def _sc_deg_body(dst_hbm, ones_hbm, zeros_hbm, out_hbm, dst_v, ones_v, acc):
    cid = lax.axis_index("c")
    sid = lax.axis_index("s")
    wid = sid * NC + cid
    pltpu.sync_copy(dst_hbm.at[wid], dst_v)
    pltpu.sync_copy(ones_hbm, ones_v)
    pltpu.sync_copy(zeros_hbm, acc.at[pl.ds(sid * STRIPE, STRIPE)])
    plsc.subcore_barrier()
    for j in range(K):
        pltpu.sync_copy(ones_v, acc.at[dst_v.at[j]], add=True)
    plsc.subcore_barrier()
    pltpu.sync_copy(acc.at[pl.ds(sid * STRIPE, STRIPE)],
                    out_hbm.at[cid, pl.ds(sid * STRIPE, STRIPE)])


def _sc_agg_body(g_hbm, src_hbm, dst_hbm, zeros_hbm, out_hbm,
                 src_v, dst_v, buf0, buf1, acc, sem0, sem1):
    cid = lax.axis_index("c")
    sid = lax.axis_index("s")
    wid = sid * NC + cid
    pltpu.sync_copy(src_hbm.at[wid], src_v)
    pltpu.sync_copy(dst_hbm.at[wid], dst_v)
    pltpu.sync_copy(zeros_hbm, acc.at[pl.ds(sid * STRIPE, STRIPE)])
    plsc.subcore_barrier()
    bufs = (buf0, buf1)
    sems = (sem0, sem1)
    descs = [None] * K
    descs[0] = pltpu.async_copy(g_hbm.at[src_v.at[0]], bufs[0], sems[0])
    for j in range(K):
        if j + 1 < K:
            descs[j + 1] = pltpu.async_copy(
                g_hbm.at[src_v.at[j + 1]], bufs[(j + 1) % 2], sems[(j + 1) % 2])
        descs[j].wait()
        pltpu.sync_copy(bufs[j % 2], acc.at[dst_v.at[j]], add=True)
    plsc.subcore_barrier()
    pltpu.sync_copy(acc.at[pl.ds(sid * STRIPE, STRIPE)],
                    out_hbm.at[cid, pl.ds(sid * STRIPE, STRIPE)])


def _make_mesh():
    return plsc.VectorSubcoreMesh(core_axis_name="c", subcore_axis_name="s")


def _sc_deg(dst3, ones16, zeros16):
    return pl.kernel(
        _sc_deg_body,
        out_type=jax.ShapeDtypeStruct((NC, NPAD, 16), jnp.float32),
        mesh=_make_mesh(),
        compiler_params=pltpu.CompilerParams(use_tc_tiling_on_sc=False),
        scratch_types=[
            pltpu.VMEM((K, CH), jnp.int32),
            pltpu.VMEM((CH, 16), jnp.float32),
            pltpu.VMEM_SHARED((NPAD, 16), jnp.float32),
        ],
    )(dst3, ones16, zeros16)


def _sc_agg(g, src3, dst3, zeros):
    d = g.shape[1]
    return pl.kernel(
        _sc_agg_body,
        out_type=jax.ShapeDtypeStruct((NC, NPAD, d), jnp.float32),
        mesh=_make_mesh(),
        compiler_params=pltpu.CompilerParams(use_tc_tiling_on_sc=False),
        scratch_types=[
            pltpu.VMEM((K, CH), jnp.int32),
            pltpu.VMEM((K, CH), jnp.int32),
            pltpu.VMEM((CH, d), jnp.float32),
            pltpu.VMEM((CH, d), jnp.float32),
            pltpu.VMEM_SHARED((NPAD, d), jnp.float32),
            pltpu.SemaphoreType.DMA,
            pltpu.SemaphoreType.DMA,
        ],
    )(g, src3, dst3, zeros)


def _rows(d):
    return pl.BlockSpec((BLK, d), lambda i: (i, 0))


def _full(r, c):
    return pl.BlockSpec((r, c), lambda i: (0, 0))


def _tc_a_body(x_ref, w_ref, p0_ref, p1_ref, gl_ref, gr_ref, dis_ref):
    deg = p0_ref[:, 0:1] + p1_ref[:, 0:1] + 1.0
    dis = lax.rsqrt(jnp.maximum(deg, 1.0))
    hw = jnp.dot(x_ref[...], w_ref[...], preferred_element_type=jnp.float32)
    g = hw * dis
    gl_ref[...] = g[:, :64]
    gr_ref[...] = g[:, 64:]
    dis_ref[...] = jnp.broadcast_to(dis, dis_ref.shape)


def _tc_a(x, w, p0, p1):
    d_in, d_out = w.shape
    return pl.pallas_call(
        _tc_a_body,
        grid=(N // BLK,),
        in_specs=[_rows(d_in), _full(d_in, d_out), _rows(16), _rows(16)],
        out_specs=[_rows(64), _rows(64), _rows(128)],
        out_shape=[jax.ShapeDtypeStruct((N, 64), jnp.float32),
                   jax.ShapeDtypeStruct((N, 64), jnp.float32),
                   jax.ShapeDtypeStruct((N, 128), jnp.float32)],
    )(x, w, p0, p1)


def _tc_comb_body(a0l_ref, a1l_ref, gl_ref, a0r_ref, a1r_ref, gr_ref,
                  dis_ref, b_ref, pre_ref, st_ref):
    i = pl.program_id(0)
    dis = dis_ref[...]
    prel = (a0l_ref[...] + a1l_ref[...] + gl_ref[...]) * dis[:, :64] + b_ref[0:1, :64]
    prer = (a0r_ref[...] + a1r_ref[...] + gr_ref[...]) * dis[:, 64:] + b_ref[0:1, 64:]
    pre = jnp.concatenate([prel, prer], axis=1)
    pre_ref[...] = pre
    @pl.when(i == 0)
    def _():
        st_ref[...] = jnp.zeros_like(st_ref)
    s = jnp.sum(pre, axis=0, keepdims=True)
    sq = jnp.sum(pre * pre, axis=0, keepdims=True)
    pad = jnp.zeros((6, pre.shape[1]), jnp.float32)
    st_ref[...] += jnp.concatenate([s, sq, pad], axis=0)


def _tc_comb(a0l, a1l, gl, a0r, a1r, gr, dis_b, b8):
    return pl.pallas_call(
        _tc_comb_body,
        grid=(N // BLK,),
        in_specs=[_rows(64)] * 6 + [_rows(128), _full(8, 128)],
        out_specs=[_rows(128), _full(8, 128)],
        out_shape=[jax.ShapeDtypeStruct((N, 128), jnp.float32),
                   jax.ShapeDtypeStruct((8, 128), jnp.float32)],
    )(a0l, a1l, gl, a0r, a1r, gr, dis_b, b8)


def _tc_bnmm_body(pre_ref, st_ref, gam_ref, bet_ref, w_ref, dis_ref, *out_refs):
    m = st_ref[0:1, :] / N
    v = st_ref[1:2, :] / N - m * m
    h = gam_ref[0:1, :] * (pre_ref[...] - m) * lax.rsqrt(v + BN_EPS) + bet_ref[0:1, :]
    h = jnp.maximum(h, 0.0)
    g = jnp.dot(h, w_ref[...], preferred_element_type=jnp.float32) * dis_ref[...]
    if len(out_refs) == 1:
        out_refs[0][...] = g
    else:
        out_refs[0][...] = g[:, :64]
        out_refs[1][...] = g[:, 64:]


def _tc_bnmm(pre, st, gam8, bet8, w, dis_b):
    d_in, d_out = w.shape
    if d_out == 128:
        out_specs = [_rows(64), _rows(64)]
        out_shape = [jax.ShapeDtypeStruct((N, 64), jnp.float32),
                     jax.ShapeDtypeStruct((N, 64), jnp.float32)]
    else:
        out_specs = _rows(d_out)
        out_shape = jax.ShapeDtypeStruct((N, d_out), jnp.float32)
    return pl.pallas_call(
        _tc_bnmm_body,
        grid=(N // BLK,),
        in_specs=[_rows(d_in), _full(8, d_in), _full(8, d_in), _full(8, d_in),
                  _full(d_in, d_out), _rows(d_out)],
        out_specs=out_specs,
        out_shape=out_shape,
    )(pre, st, gam8, bet8, w, dis_b)


def _tc_final_body(a0_ref, a1_ref, g_ref, dis_ref, b_ref, out_ref):
    out_ref[...] = ((a0_ref[...] + a1_ref[...] + g_ref[...]) * dis_ref[...]
                    + b_ref[0:1, :])


def _tc_final(a0, a1, g, dis_b, b8):
    d = g.shape[1]
    return pl.pallas_call(
        _tc_final_body,
        grid=(N // BLK,),
        in_specs=[_rows(d), _rows(d), _rows(d), _rows(d), _full(8, d)],
        out_specs=_rows(d),
        out_shape=jax.ShapeDtypeStruct((N, d), jnp.float32),
    )(a0, a1, g, dis_b, b8)


def _row8(v):
    return jnp.broadcast_to(v.reshape(1, -1), (8, v.shape[0]))


def kernel(x, edge_index, W1, b1, g1, be1, W2, b2, g2, be2, W3, b3):
    pad = EPAD - E
    src3 = jnp.concatenate(
        [edge_index[0], jnp.zeros((pad,), jnp.int32)]).reshape(NW, K, CH)
    dst3 = jnp.concatenate(
        [edge_index[1], jnp.full((pad,), N, jnp.int32)]).reshape(NW, K, CH)
    zeros64 = jnp.zeros((STRIPE, 64), jnp.float32)
    zeros16 = jnp.zeros((STRIPE, 16), jnp.float32)
    ones16 = jnp.ones((CH, 16), jnp.float32)

    degp = _sc_deg(dst3, ones16, zeros16)
    p0 = degp[0, :N, :]
    p1 = degp[1, :N, :]

    gl, gr, dis_b = _tc_a(x, W1, p0, p1)
    al = _sc_agg(gl, src3, dst3, zeros64)
    ar = _sc_agg(gr, src3, dst3, zeros64)
    pre, st = _tc_comb(al[0, :N], al[1, :N], gl, ar[0, :N], ar[1, :N], gr,
                       dis_b, _row8(b1))
    gl, gr = _tc_bnmm(pre, st, _row8(g1), _row8(be1), W2, dis_b)
    al = _sc_agg(gl, src3, dst3, zeros64)
    ar = _sc_agg(gr, src3, dst3, zeros64)
    pre, st = _tc_comb(al[0, :N], al[1, :N], gl, ar[0, :N], ar[1, :N], gr,
                       dis_b, _row8(b2))
    dis64 = dis_b[:, :64]
    gx = _tc_bnmm(pre, st, _row8(g2), _row8(be2), W3, dis64)
    a = _sc_agg(gx, src3, dst3, zeros64)
    return _tc_final(a[0, :N], a[1, :N], gx, dis64, _row8(b3))


# speedup vs baseline: 12.8534x; 1.0144x over previous
import jax
import jax.numpy as jnp
from jax import lax
from jax.experimental import pallas as pl
from jax.experimental.pallas import tpu as pltpu
from jax.experimental.pallas import tpu_sc as plsc

N = 10000
NPAD = 10112
E = 320000
CH = 128
NC, NS = 2, 16
NW = NC * NS
K = -(-E // (NW * CH))
EPT = K * CH
EPAD = EPT * NW
STRIPE = NPAD // NS
BN_EPS = 1e-5
BLK = 1000


def _sc_deg_body(dst_hbm, ones_hbm, zeros_hbm, out_hbm, dst_v, ones_v, acc):
    cid = lax.axis_index("c")
    sid = lax.axis_index("s")
    wid = sid * NC + cid
    pltpu.sync_copy(dst_hbm.at[wid], dst_v)
    pltpu.sync_copy(ones_hbm, ones_v)
    pltpu.sync_copy(zeros_hbm, acc.at[pl.ds(sid * STRIPE, STRIPE)])
    plsc.subcore_barrier()
    for j in range(K):
        pltpu.sync_copy(ones_v, acc.at[dst_v.at[j]], add=True)
    plsc.subcore_barrier()
    pltpu.sync_copy(acc.at[pl.ds(sid * STRIPE, STRIPE)],
                    out_hbm.at[cid, pl.ds(sid * STRIPE, STRIPE)])


def _sc_agg_body(g_hbm, src_hbm, dst_hbm, zeros_hbm, out_hbm,
                 src_v, dst_v, buf0, buf1, acc, sem0, sem1):
    cid = lax.axis_index("c")
    sid = lax.axis_index("s")
    wid = sid * NC + cid
    pltpu.sync_copy(src_hbm.at[wid], src_v)
    pltpu.sync_copy(dst_hbm.at[wid], dst_v)
    pltpu.sync_copy(zeros_hbm, acc.at[pl.ds(sid * STRIPE, STRIPE)])
    plsc.subcore_barrier()
    bufs = (buf0, buf1)
    sems = (sem0, sem1)
    descs = [None] * K
    descs[0] = pltpu.async_copy(g_hbm.at[src_v.at[0]], bufs[0], sems[0])
    for j in range(K):
        if j + 1 < K:
            descs[j + 1] = pltpu.async_copy(
                g_hbm.at[src_v.at[j + 1]], bufs[(j + 1) % 2], sems[(j + 1) % 2])
        descs[j].wait()
        pltpu.sync_copy(bufs[j % 2], acc.at[dst_v.at[j]], add=True)
    plsc.subcore_barrier()
    pltpu.sync_copy(acc.at[pl.ds(sid * STRIPE, STRIPE)],
                    out_hbm.at[cid, pl.ds(sid * STRIPE, STRIPE)])


def _make_mesh():
    return plsc.VectorSubcoreMesh(core_axis_name="c", subcore_axis_name="s")


def _sc_deg(dst3, ones16, zeros16):
    return pl.kernel(
        _sc_deg_body,
        out_type=jax.ShapeDtypeStruct((NC, NPAD, 16), jnp.float32),
        mesh=_make_mesh(),
        compiler_params=pltpu.CompilerParams(use_tc_tiling_on_sc=False),
        scratch_types=[
            pltpu.VMEM((K, CH), jnp.int32),
            pltpu.VMEM((CH, 16), jnp.float32),
            pltpu.VMEM_SHARED((NPAD, 16), jnp.float32),
        ],
    )(dst3, ones16, zeros16)


def _sc_agg(g, src3, dst3, zeros):
    d = g.shape[1]
    return pl.kernel(
        _sc_agg_body,
        out_type=jax.ShapeDtypeStruct((NC, NPAD, d), jnp.float32),
        mesh=_make_mesh(),
        compiler_params=pltpu.CompilerParams(use_tc_tiling_on_sc=False),
        scratch_types=[
            pltpu.VMEM((K, CH), jnp.int32),
            pltpu.VMEM((K, CH), jnp.int32),
            pltpu.VMEM((CH, d), jnp.float32),
            pltpu.VMEM((CH, d), jnp.float32),
            pltpu.VMEM_SHARED((NPAD, d), jnp.float32),
            pltpu.SemaphoreType.DMA,
            pltpu.SemaphoreType.DMA,
        ],
    )(g, src3, dst3, zeros)


def _rows(d):
    return pl.BlockSpec((BLK, d), lambda i: (i, 0))


def _full(r, c):
    return pl.BlockSpec((r, c), lambda i: (0, 0))


def _tc_a_body(x_ref, w_ref, p0_ref, p1_ref, gl_ref, gr_ref, dis_ref):
    deg = p0_ref[:, 0:1] + p1_ref[:, 0:1] + 1.0
    dis = lax.rsqrt(jnp.maximum(deg, 1.0))
    hw = jnp.dot(x_ref[...], w_ref[...], preferred_element_type=jnp.float32)
    g = hw * dis
    gl_ref[...] = g[:, :64]
    gr_ref[...] = g[:, 64:]
    dis_ref[...] = jnp.broadcast_to(dis, dis_ref.shape)


def _tc_a(x, w, p0, p1):
    d_in, d_out = w.shape
    return pl.pallas_call(
        _tc_a_body,
        grid=(N // BLK,),
        in_specs=[_rows(d_in), _full(d_in, d_out), _rows(16), _rows(16)],
        out_specs=[_rows(64), _rows(64), _rows(128)],
        out_shape=[jax.ShapeDtypeStruct((N, 64), jnp.float32),
                   jax.ShapeDtypeStruct((N, 64), jnp.float32),
                   jax.ShapeDtypeStruct((N, 128), jnp.float32)],
    )(x, w, p0, p1)


def _tc_comb_body(a0l_ref, a1l_ref, gl_ref, a0r_ref, a1r_ref, gr_ref,
                  dis_ref, b_ref, pre_ref, st_ref):
    i = pl.program_id(0)
    dis = dis_ref[...]
    prel = (a0l_ref[...] + a1l_ref[...] + gl_ref[...]) * dis[:, :64] + b_ref[0:1, :64]
    prer = (a0r_ref[...] + a1r_ref[...] + gr_ref[...]) * dis[:, 64:] + b_ref[0:1, 64:]
    pre = jnp.concatenate([prel, prer], axis=1)
    pre_ref[...] = pre
    @pl.when(i == 0)
    def _():
        st_ref[...] = jnp.zeros_like(st_ref)
    s = jnp.sum(pre, axis=0, keepdims=True)
    sq = jnp.sum(pre * pre, axis=0, keepdims=True)
    pad = jnp.zeros((6, pre.shape[1]), jnp.float32)
    st_ref[...] += jnp.concatenate([s, sq, pad], axis=0)


def _tc_comb(a0l, a1l, gl, a0r, a1r, gr, dis_b, b8):
    return pl.pallas_call(
        _tc_comb_body,
        grid=(N // BLK,),
        in_specs=[_rows(64)] * 6 + [_rows(128), _full(8, 128)],
        out_specs=[_rows(128), _full(8, 128)],
        out_shape=[jax.ShapeDtypeStruct((N, 128), jnp.float32),
                   jax.ShapeDtypeStruct((8, 128), jnp.float32)],
    )(a0l, a1l, gl, a0r, a1r, gr, dis_b, b8)


def _tc_bnmm_body(pre_ref, st_ref, gam_ref, bet_ref, w_ref, dis_ref, *out_refs):
    m = st_ref[0:1, :] / N
    v = st_ref[1:2, :] / N - m * m
    h = gam_ref[0:1, :] * (pre_ref[...] - m) * lax.rsqrt(v + BN_EPS) + bet_ref[0:1, :]
    h = jnp.maximum(h, 0.0)
    g = jnp.dot(h, w_ref[...], preferred_element_type=jnp.float32) * dis_ref[...]
    if len(out_refs) == 1:
        out_refs[0][...] = g
    else:
        out_refs[0][...] = g[:, :64]
        out_refs[1][...] = g[:, 64:]


def _tc_bnmm(pre, st, gam8, bet8, w, dis_b):
    d_in, d_out = w.shape
    if d_out == 128:
        out_specs = [_rows(64), _rows(64)]
        out_shape = [jax.ShapeDtypeStruct((N, 64), jnp.float32),
                     jax.ShapeDtypeStruct((N, 64), jnp.float32)]
    else:
        out_specs = _rows(d_out)
        out_shape = jax.ShapeDtypeStruct((N, d_out), jnp.float32)
    return pl.pallas_call(
        _tc_bnmm_body,
        grid=(N // BLK,),
        in_specs=[_rows(d_in), _full(8, d_in), _full(8, d_in), _full(8, d_in),
                  _full(d_in, d_out), _rows(d_out)],
        out_specs=out_specs,
        out_shape=out_shape,
    )(pre, st, gam8, bet8, w, dis_b)


def _tc_final_body(a0_ref, a1_ref, g_ref, dis_ref, b_ref, out_ref):
    out_ref[...] = ((a0_ref[...] + a1_ref[...] + g_ref[...]) * dis_ref[...]
                    + b_ref[0:1, :])


def _tc_final(a0, a1, g, dis_b, b8):
    d = g.shape[1]
    return pl.pallas_call(
        _tc_final_body,
        grid=(N // BLK,),
        in_specs=[_rows(d), _rows(d), _rows(d), _rows(d), _full(8, d)],
        out_specs=_rows(d),
        out_shape=jax.ShapeDtypeStruct((N, d), jnp.float32),
    )(a0, a1, g, dis_b, b8)


def _row8(v):
    return jnp.broadcast_to(v.reshape(1, -1), (8, v.shape[0]))


def kernel(x, edge_index, W1, b1, g1, be1, W2, b2, g2, be2, W3, b3):
    pad = EPAD - E
    src3 = jnp.concatenate(
        [edge_index[0], jnp.zeros((pad,), jnp.int32)]).reshape(NW, K, CH)
    pad_dst = N + (jnp.arange(pad, dtype=jnp.int32) % (NPAD - N))
    dst3 = jnp.concatenate([edge_index[1], pad_dst]).reshape(NW, K, CH)
    zeros64 = jnp.zeros((STRIPE, 64), jnp.float32)
    zeros16 = jnp.zeros((STRIPE, 16), jnp.float32)
    ones16 = jnp.ones((CH, 16), jnp.float32)

    degp = _sc_deg(dst3, ones16, zeros16)
    p0 = degp[0, :N, :]
    p1 = degp[1, :N, :]

    gl, gr, dis_b = _tc_a(x, W1, p0, p1)
    al = _sc_agg(gl, src3, dst3, zeros64)
    ar = _sc_agg(gr, src3, dst3, zeros64)
    pre, st = _tc_comb(al[0, :N], al[1, :N], gl, ar[0, :N], ar[1, :N], gr,
                       dis_b, _row8(b1))
    gl, gr = _tc_bnmm(pre, st, _row8(g1), _row8(be1), W2, dis_b)
    al = _sc_agg(gl, src3, dst3, zeros64)
    ar = _sc_agg(gr, src3, dst3, zeros64)
    pre, st = _tc_comb(al[0, :N], al[1, :N], gl, ar[0, :N], ar[1, :N], gr,
                       dis_b, _row8(b2))
    dis64 = dis_b[:, :64]
    gx = _tc_bnmm(pre, st, _row8(g2), _row8(be2), W3, dis64)
    a = _sc_agg(gx, src3, dst3, zeros64)
    return _tc_final(a[0, :N], a[1, :N], gx, dis64, _row8(b3))
